# initial kernel scaffold (unmeasured)
import jax
import jax.numpy as jnp
from jax import lax
from jax.experimental import pallas as pl
from jax.experimental.pallas import tpu as pltpu

N_GLOBAL = 4096
EPS = 1e-5
CHUNK = 512


def kernel(x, gamma, beta):
    m, n = x.shape
    nchunks = m // CHUNK

    def body(x_hbm, g_ref, b_ref, o_ref, stage, stats, rstats,
             copy_sems, send_sem, recv_sem):
        my_x = lax.axis_index("x")
        my_y = lax.axis_index("y")
        peer = (my_x, 1 - my_y)

        barrier = pltpu.get_barrier_semaphore()
        pl.semaphore_signal(barrier, inc=1, device_id=peer,
                            device_id_type=pl.DeviceIdType.MESH)
        pl.semaphore_wait(barrier, 1)

        def make_load(i, slot):
            return pltpu.make_async_copy(
                x_hbm.at[pl.ds(i * CHUNK, CHUNK), :],
                stage.at[slot],
                copy_sems.at[slot],
            )

        cps = {}
        cps[0] = make_load(0, 0)
        cps[0].start()
        for i in range(nchunks):
            slot = i % 2
            if i + 1 < nchunks:
                cps[(i + 1) % 2] = make_load(i + 1, (i + 1) % 2)
                cps[(i + 1) % 2].start()
            cps[slot].wait()
            c = stage[slot]
            s1 = jnp.sum(c, axis=1, keepdims=True)
            s2 = jnp.sum(c * c, axis=1, keepdims=True)
            stats[pl.ds(i * CHUNK, CHUNK), :] = jnp.concatenate(
                [s1, s2], axis=1)

        rdma = pltpu.make_async_remote_copy(
            src_ref=stats,
            dst_ref=rstats,
            send_sem=send_sem,
            recv_sem=recv_sem,
            device_id=peer,
            device_id_type=pl.DeviceIdType.MESH,
        )
        rdma.start()
        rdma.wait()

        g = g_ref[...]
        b = b_ref[...]

        cps[0] = make_load(0, 0)
        cps[0].start()
        for i in range(nchunks):
            slot = i % 2
            if i + 1 < nchunks:
                cps[(i + 1) % 2] = make_load(i + 1, (i + 1) % 2)
                cps[(i + 1) % 2].start()
            cps[slot].wait()
            c = stage[slot]
            st = (stats[pl.ds(i * CHUNK, CHUNK), :]
                  + rstats[pl.ds(i * CHUNK, CHUNK), :])
            mean = st[:, 0:1] * (1.0 / N_GLOBAL)
            var = st[:, 1:2] * (1.0 / N_GLOBAL) - mean * mean
            inv = lax.rsqrt(var + EPS)
            o_ref[pl.ds(i * CHUNK, CHUNK), :] = (
                (c - mean) * (inv * g) + b).astype(o_ref.dtype)

    return pl.pallas_call(
        body,
        out_shape=jax.ShapeDtypeStruct((m, n), jnp.bfloat16),
        in_specs=[
            pl.BlockSpec(memory_space=pltpu.ANY),
            pl.BlockSpec(memory_space=pltpu.VMEM),
            pl.BlockSpec(memory_space=pltpu.VMEM),
        ],
        out_specs=pl.BlockSpec(memory_space=pltpu.VMEM),
        scratch_shapes=[
            pltpu.VMEM((2, CHUNK, n), jnp.float32),
            pltpu.VMEM((m, 2), jnp.float32),
            pltpu.VMEM((m, 2), jnp.float32),
            pltpu.SemaphoreType.DMA((2,)),
            pltpu.SemaphoreType.DMA,
            pltpu.SemaphoreType.DMA,
        ],
        compiler_params=pltpu.CompilerParams(collective_id=0),
    )(x, gamma.reshape(1, n), beta.reshape(1, n))


# baseline (device time: 100267 ns/iter reference)
import jax
import jax.numpy as jnp
from jax import lax
from jax.experimental import pallas as pl
from jax.experimental.pallas import tpu as pltpu

N_GLOBAL = 4096
EPS = 1e-5
CHUNK = 512


def kernel(x, gamma, beta):
    m, n = x.shape
    nchunks = m // CHUNK

    def body(x_hbm, g_ref, b_ref, o_ref, stage, stats, rstats,
             copy_sems, send_sem, recv_sem):
        my_x = lax.axis_index("x")
        my_y = lax.axis_index("y")
        peer = (my_x, 1 - my_y)

        barrier = pltpu.get_barrier_semaphore()
        pl.semaphore_signal(barrier, inc=1, device_id=peer,
                            device_id_type=pl.DeviceIdType.MESH)
        pl.semaphore_wait(barrier, 1)

        def make_load(i, slot):
            return pltpu.make_async_copy(
                x_hbm.at[pl.ds(i * CHUNK, CHUNK), :],
                stage.at[slot],
                copy_sems.at[slot],
            )

        cps = {}
        cps[0] = make_load(0, 0)
        cps[0].start()
        for i in range(nchunks):
            slot = i % 2
            if i + 1 < nchunks:
                cps[(i + 1) % 2] = make_load(i + 1, (i + 1) % 2)
                cps[(i + 1) % 2].start()
            cps[slot].wait()
            c = stage[slot]
            s1 = jnp.sum(c, axis=1, keepdims=True)
            s2 = jnp.sum(c * c, axis=1, keepdims=True)
            stats[pl.ds(i * CHUNK, CHUNK), :] = jnp.concatenate(
                [s1, s2], axis=1)

        rdma = pltpu.make_async_remote_copy(
            src_ref=stats,
            dst_ref=rstats,
            send_sem=send_sem,
            recv_sem=recv_sem,
            device_id=peer,
            device_id_type=pl.DeviceIdType.MESH,
        )
        rdma.start()
        rdma.wait()

        g = g_ref[...]
        b = b_ref[...]

        cps[0] = make_load(0, 0)
        cps[0].start()
        for i in range(nchunks):
            slot = i % 2
            if i + 1 < nchunks:
                cps[(i + 1) % 2] = make_load(i + 1, (i + 1) % 2)
                cps[(i + 1) % 2].start()
            cps[slot].wait()
            c = stage[slot]
            st = (stats[pl.ds(i * CHUNK, CHUNK), :]
                  + rstats[pl.ds(i * CHUNK, CHUNK), :])
            mean = st[:, 0:1] * (1.0 / N_GLOBAL)
            var = st[:, 1:2] * (1.0 / N_GLOBAL) - mean * mean
            inv = lax.rsqrt(var + EPS)
            o_ref[pl.ds(i * CHUNK, CHUNK), :] = (
                (c - mean) * (inv * g) + b).astype(o_ref.dtype)

    return pl.pallas_call(
        body,
        out_shape=jax.ShapeDtypeStruct((m, n), jnp.bfloat16),
        in_specs=[
            pl.BlockSpec(memory_space=pl.ANY),
            pl.BlockSpec(memory_space=pltpu.VMEM),
            pl.BlockSpec(memory_space=pltpu.VMEM),
        ],
        out_specs=pl.BlockSpec(memory_space=pltpu.VMEM),
        scratch_shapes=[
            pltpu.VMEM((2, CHUNK, n), jnp.float32),
            pltpu.VMEM((m, 2), jnp.float32),
            pltpu.VMEM((m, 2), jnp.float32),
            pltpu.SemaphoreType.DMA((2,)),
            pltpu.SemaphoreType.DMA,
            pltpu.SemaphoreType.DMA,
        ],
        compiler_params=pltpu.CompilerParams(
            collective_id=0,
            vmem_limit_bytes=60 * 1024 * 1024,
        ),
    )(x, gamma.reshape(1, n), beta.reshape(1, n))


# device time: 86145 ns/iter; 1.1639x vs baseline; 1.1639x over previous
import jax
import jax.numpy as jnp
from jax import lax
from jax.experimental import pallas as pl
from jax.experimental.pallas import tpu as pltpu

N_GLOBAL = 4096
EPS = 1e-5
CHUNK = 512
NBUF = 3


def kernel(x, gamma, beta):
    m, n = x.shape
    nchunks = m // CHUNK

    def body(x_hbm, g_ref, b_ref, o_hbm, stage, xb, out_stage, stats, rstats,
             in_sems, out_sems, send_sem, recv_sem):
        my_x = lax.axis_index("x")
        my_y = lax.axis_index("y")
        peer = (my_x, 1 - my_y)

        barrier = pltpu.get_barrier_semaphore()
        pl.semaphore_signal(barrier, inc=1, device_id=peer,
                            device_id_type=pl.DeviceIdType.MESH)
        pl.semaphore_wait(barrier, 1)

        def make_load(i, slot):
            return pltpu.make_async_copy(
                x_hbm.at[pl.ds(i * CHUNK, CHUNK), :],
                stage.at[slot],
                in_sems.at[slot],
            )

        def make_store(i, slot):
            return pltpu.make_async_copy(
                out_stage.at[slot],
                o_hbm.at[pl.ds(i * CHUNK, CHUNK), :],
                out_sems.at[slot],
            )

        loads = {}
        for j in range(min(NBUF, nchunks)):
            loads[j] = make_load(j, j)
            loads[j].start()
        for i in range(nchunks):
            slot = i % NBUF
            loads[slot].wait()
            rows = pl.ds(i * CHUNK, CHUNK)
            c = stage[slot]
            xb[rows, :] = c.astype(jnp.bfloat16)
            s1 = jnp.sum(c, axis=1, keepdims=True)
            s2 = jnp.sum(c * c, axis=1, keepdims=True)
            stats[rows, :] = jnp.concatenate([s1, s2], axis=1)
            if i + NBUF < nchunks:
                loads[slot] = make_load(i + NBUF, slot)
                loads[slot].start()

        rdma = pltpu.make_async_remote_copy(
            src_ref=stats,
            dst_ref=rstats,
            send_sem=send_sem,
            recv_sem=recv_sem,
            device_id=peer,
            device_id_type=pl.DeviceIdType.MESH,
        )
        rdma.start()
        rdma.wait()

        g = g_ref[...]
        b = b_ref[...]

        stores = {}
        for i in range(nchunks):
            slot = i % 2
            if i >= 2:
                stores[slot].wait()
            rows = pl.ds(i * CHUNK, CHUNK)
            st = stats[rows, :] + rstats[rows, :]
            mean = st[:, 0:1] * (1.0 / N_GLOBAL)
            var = st[:, 1:2] * (1.0 / N_GLOBAL) - mean * mean
            inv = lax.rsqrt(var + EPS)
            c = xb[rows, :].astype(jnp.float32)
            out_stage[slot, :, :] = ((c - mean) * (inv * g) + b).astype(
                jnp.bfloat16)
            stores[slot] = make_store(i, slot)
            stores[slot].start()
        stores[0].wait()
        stores[1].wait()

    return pl.pallas_call(
        body,
        out_shape=jax.ShapeDtypeStruct((m, n), jnp.bfloat16),
        in_specs=[
            pl.BlockSpec(memory_space=pl.ANY),
            pl.BlockSpec(memory_space=pltpu.VMEM),
            pl.BlockSpec(memory_space=pltpu.VMEM),
        ],
        out_specs=pl.BlockSpec(memory_space=pl.ANY),
        scratch_shapes=[
            pltpu.VMEM((NBUF, CHUNK, n), jnp.float32),
            pltpu.VMEM((m, n), jnp.bfloat16),
            pltpu.VMEM((2, CHUNK, n), jnp.bfloat16),
            pltpu.VMEM((m, 2), jnp.float32),
            pltpu.VMEM((m, 2), jnp.float32),
            pltpu.SemaphoreType.DMA((NBUF,)),
            pltpu.SemaphoreType.DMA((2,)),
            pltpu.SemaphoreType.DMA,
            pltpu.SemaphoreType.DMA,
        ],
        compiler_params=pltpu.CompilerParams(
            collective_id=0,
            vmem_limit_bytes=60 * 1024 * 1024,
        ),
    )(x, gamma.reshape(1, n), beta.reshape(1, n))
